# baseline (speedup 1.0000x reference)
import functools

import jax
import jax.numpy as jnp
from jax import lax
from jax.experimental import pallas as pl
from jax.experimental.pallas import tpu as pltpu
from jax.experimental.pallas import tpu_sc as plsc

_NC = 2
_NS = 16
_NW = _NC * _NS
_CHUNK = 128
_LANES = 16


def _mm_bias(x, w, b, rows_per_block=1000):
    n, k = x.shape
    m = w.shape[1]
    grid = n // rows_per_block

    def body(x_ref, w_ref, b_ref, o_ref):
        o_ref[...] = (
            jnp.dot(x_ref[...], w_ref[...], preferred_element_type=jnp.float32)
            + b_ref[...]
        )

    return pl.pallas_call(
        body,
        grid=(grid,),
        in_specs=[
            pl.BlockSpec((rows_per_block, k), lambda i: (i, 0)),
            pl.BlockSpec((k, m), lambda i: (0, 0)),
            pl.BlockSpec((1, m), lambda i: (0, 0)),
        ],
        out_specs=pl.BlockSpec((rows_per_block, m), lambda i: (i, 0)),
        out_shape=jax.ShapeDtypeStruct((n, m), jnp.float32),
    )(x, w, b.reshape(1, m))


def _relu_sum_mm_bias(p, w, b, n, rows_per_block=1000):
    k = p.shape[2]
    m = w.shape[1]
    grid = n // rows_per_block

    def body(p_ref, w_ref, b_ref, o_ref):
        h = jnp.maximum(p_ref[0] + p_ref[1], 0.0)
        o_ref[...] = (
            jnp.dot(h, w_ref[...], preferred_element_type=jnp.float32)
            + b_ref[...]
        )

    return pl.pallas_call(
        body,
        grid=(grid,),
        in_specs=[
            pl.BlockSpec((2, rows_per_block, k), lambda i: (0, i, 0)),
            pl.BlockSpec((k, m), lambda i: (0, 0)),
            pl.BlockSpec((1, m), lambda i: (0, 0)),
        ],
        out_specs=pl.BlockSpec((rows_per_block, m), lambda i: (i, 0)),
        out_shape=jax.ShapeDtypeStruct((n, m), jnp.float32),
    )(p, w, b.reshape(1, m))


@functools.lru_cache(maxsize=None)
def _make_agg(n, d, c0, c1):
    rows_per_tile = -(-n // (_NS * 8)) * 8
    n_pad = rows_per_tile * _NS
    d_vecs = d // _LANES
    h0, h1 = c0 // 6, c1 // 6

    pieces = []
    off = 0
    while off < rows_per_tile:
        sz = min(_CHUNK, rows_per_tile - off)
        pieces.append((off, sz))
        off += sz

    mesh = plsc.VectorSubcoreMesh(core_axis_name="c", subcore_axis_name="s")

    @functools.partial(
        pl.kernel,
        out_type=jax.ShapeDtypeStruct((_NC, n_pad, d), jnp.float32),
        mesh=mesh,
        scratch_types=[
            pltpu.VMEM((2, _CHUNK), jnp.int32),
            pltpu.VMEM((3, _CHUNK), jnp.int32),
            pltpu.VMEM((_CHUNK,), jnp.float32),
            pltpu.VMEM((3, _CHUNK, d), jnp.float32),
            pltpu.VMEM_SHARED((n_pad, d), jnp.float32),
            pltpu.SemaphoreType.DMA,
            pltpu.SemaphoreType.DMA,
            pltpu.SemaphoreType.DMA,
            pltpu.SemaphoreType.DMA,
            pltpu.SemaphoreType.DMA,
            pltpu.SemaphoreType.DMA,
            pltpu.SemaphoreType.DMA,
            pltpu.SemaphoreType.DMA,
            pltpu.SemaphoreType.DMA,
            pltpu.SemaphoreType.DMA,
            pltpu.SemaphoreType.DMA,
            pltpu.SemaphoreType.DMA,
        ],
    )
    def agg(hw_hbm, src_hbm, dst_hbm, ew_hbm, out_hbm,
            src_v, dst_v, ew_v, rows_v, acc_sh,
            sg0, sg1, sg2, sc0, sc1, sc2, ss0, ss1, sd0, sd1, sd2, se):
        SG = (sg0, sg1, sg2)
        SCT = (sc0, sc1, sc2)
        SS = (ss0, ss1)
        SD = (sd0, sd1, sd2)
        cid = lax.axis_index("c")
        sid = lax.axis_index("s")
        base = jnp.where(cid == 0, sid * c0, _NS * c0 + sid * c1)
        my_h = jnp.where(cid == 0, h0, h1)
        row0 = sid * rows_per_tile

        def src_fetch(g, p):
            pltpu.async_copy(src_hbm.at[base + g], src_v.at[p], SS[p])

        def src_wait(g, p):
            pltpu.make_async_copy(src_hbm.at[base + g], src_v.at[p],
                                  SS[p]).wait()

        def dst_fetch(g, q):
            pltpu.async_copy(dst_hbm.at[base + g], dst_v.at[q], SD[q])

        def dst_wait(g, q):
            pltpu.make_async_copy(dst_hbm.at[base + g], dst_v.at[q],
                                  SD[q]).wait()

        def ew_fetch(g):
            pltpu.async_copy(ew_hbm.at[base + g], ew_v, se)

        def ew_wait(g):
            pltpu.make_async_copy(ew_hbm.at[base + g], ew_v, se).wait()

        def rows_start(p, m):
            pltpu.async_copy(hw_hbm.at[src_v.at[p]], rows_v.at[m], SG[m])

        def rows_wait(p, m):
            pltpu.make_async_copy(hw_hbm.at[src_v.at[p]], rows_v.at[m],
                                  SG[m]).wait()

        def scatter_start(m):
            pltpu.async_copy(rows_v.at[m], acc_sh.at[dst_v.at[m]], SCT[m],
                             add=True)

        def scatter_wait(m):
            pltpu.make_async_copy(rows_v.at[m], acc_sh.at[dst_v.at[m]],
                                  SCT[m]).wait()

        def scale(m):
            buf = rows_v.at[m]

            def grp(gg, c2):
                wv = ew_v[pl.ds(gg * _LANES, _LANES)]
                for lane in range(_LANES):
                    w = wv[lane]
                    row = gg * _LANES + lane
                    for j in range(d_vecs):
                        sl = pl.ds(j * _LANES, _LANES)
                        buf[row, sl] = buf[row, sl] * w
                return c2
            lax.fori_loop(0, _CHUNK // _LANES, grp, 0)

        def zero_row(i, carry):
            for j in range(d_vecs):
                rows_v[0, i, pl.ds(j * _LANES, _LANES)] = jnp.zeros(
                    (_LANES,), jnp.float32)
            return carry
        lax.fori_loop(0, _CHUNK, zero_row, 0)
        for poff, psz in pieces:
            pltpu.sync_copy(rows_v.at[0].at[pl.ds(0, psz)],
                            acc_sh.at[pl.ds(row0 + poff, psz)])
        plsc.subcore_barrier()

        src_fetch(0, 0)
        src_fetch(1, 1)
        dst_fetch(0, 0)
        ew_fetch(0)
        src_wait(0, 0)
        rows_start(0, 0)

        def sixpack(h, carry):
            gb = 6 * h
            for m6 in range(6):
                g = gb + m6
                m = m6 % 3
                mn = (m6 + 1) % 3
                p = m6 % 2
                pn = (m6 + 1) % 2
                if m6 >= 2:
                    scatter_wait(mn)
                else:
                    @pl.when(h > 0)
                    def _a():
                        scatter_wait(mn)
                if m6 < 5:
                    dst_fetch(g + 1, mn)
                    src_wait(g + 1, pn)
                    rows_start(pn, mn)
                else:
                    @pl.when(h + 1 < my_h)
                    def _b():
                        dst_fetch(g + 1, mn)
                        src_wait(g + 1, pn)
                        rows_start(pn, mn)
                rows_wait(p, m)
                ew_wait(g)
                dst_wait(g, m)
                scale(m)
                if m6 < 5:
                    ew_fetch(g + 1)
                else:
                    @pl.when(h + 1 < my_h)
                    def _d():
                        ew_fetch(g + 1)
                scatter_start(m)
                if m6 < 4:
                    src_fetch(g + 2, p)
                else:
                    @pl.when(h + 1 < my_h)
                    def _e():
                        src_fetch(g + 2, p)
            return carry
        lax.fori_loop(0, my_h, sixpack, 0)
        scatter_wait(1)
        scatter_wait(2)
        plsc.subcore_barrier()

        for poff, psz in pieces:
            pltpu.sync_copy(acc_sh.at[pl.ds(row0 + poff, psz)],
                            out_hbm.at[cid, pl.ds(row0 + poff, psz)])

    return agg


_CORE0_FRAC = 0.5


def kernel(feats, edge_index, edge_weight, W1, b1, W2, b2, Wmu, bmu, Wlv, blv):
    n, d = feats.shape
    e = edge_index.shape[1]
    n_chunks = -(-e // (_NS * _CHUNK * 6)) * 6
    c0 = max(6, int(round(n_chunks * _CORE0_FRAC / 6)) * 6)
    c1 = n_chunks - c0
    q_tot = _NS * n_chunks
    e_pad = q_tot * _CHUNK

    shp = (q_tot, _CHUNK)
    fill = (jnp.arange(e_pad - e, dtype=jnp.int32) * 8) % n
    src = jnp.concatenate([edge_index[0], fill]).reshape(shp)
    dst = jnp.concatenate([edge_index[1], fill]).reshape(shp)
    ew = jnp.pad(edge_weight, (0, e_pad - e)).reshape(shp)

    agg = _make_agg(n, d, c0, c1)

    hw1 = _mm_bias(feats, W1, b1)
    p1 = agg(hw1, src, dst, ew)
    hw2 = _relu_sum_mm_bias(p1, W2, b2, n)
    p2 = agg(hw2, src, dst, ew)
    wcat = jnp.concatenate([Wmu, Wlv], axis=1)
    bcat = jnp.concatenate([bmu, blv])
    mv = _relu_sum_mm_bias(p2, wcat, bcat, n)
    l = Wmu.shape[1]
    return mv[:, :l], mv[:, l:]
